# baseline (speedup 1.0000x reference)
import functools

import jax
import jax.numpy as jnp
from jax import lax
from jax.experimental import pallas as pl
from jax.experimental.pallas import tpu as pltpu
from jax.experimental.pallas import tpu_sc as plsc

_NC = 2
_NS = 16
_NW = _NC * _NS
_CH = 16


def kernel(x, pe):
    B, S, D = x.shape
    seq_per_w = S // _NW
    chunk_elems = _CH * D
    xf = x.reshape(B * S * D)
    pef = pe.reshape(S * D)

    mesh = plsc.VectorSubcoreMesh(core_axis_name="c", subcore_axis_name="s")

    @functools.partial(
        pl.kernel,
        mesh=mesh,
        out_type=jax.ShapeDtypeStruct((B * S * D,), jnp.float32),
        scratch_types=[
            pltpu.VMEM((chunk_elems,), jnp.float32),
            pltpu.VMEM((chunk_elems,), jnp.float32),
        ],
    )
    def sc_add(xf_hbm, pe_hbm, out_hbm, pe_v, acc_v):
        wid = lax.axis_index("s") * _NC + lax.axis_index("c")
        s_base = wid * seq_per_w

        def chunk(i, carry):
            s0 = (s_base + i * _CH) * D
            pltpu.sync_copy(pe_hbm.at[pl.ds(s0, chunk_elems)], pe_v)

            def per_batch(b, c2):
                a = b * (S * D) + s0
                pltpu.sync_copy(xf_hbm.at[pl.ds(a, chunk_elems)], acc_v)

                @plsc.parallel_loop(0, chunk_elems // 16, unroll=8)
                def add_vec(k):
                    off = k * 16
                    plsc.addupdate(acc_v.at[pl.ds(off, 16)],
                                   pe_v[pl.ds(off, 16)])
                pltpu.sync_copy(acc_v, out_hbm.at[pl.ds(a, chunk_elems)])
                return c2

            lax.fori_loop(0, B, per_batch, 0)
            return carry

        lax.fori_loop(0, seq_per_w // _CH, chunk, 0)

    return sc_add(xf, pef).reshape(B, S, D)


# speedup vs baseline: 4.9773x; 4.9773x over previous
import jax
import jax.numpy as jnp
from jax.experimental import pallas as pl
from jax.experimental.pallas import tpu as pltpu


def _body(x_ref, pe_ref, o_ref):
    o_ref[...] = x_ref[...] + pe_ref[...][None]


def kernel(x, pe):
    B, S, D = x.shape
    BS = 512
    return pl.pallas_call(
        _body,
        grid=(S // BS, B),
        compiler_params=pltpu.CompilerParams(
            dimension_semantics=("parallel", "arbitrary"),
        ),
        in_specs=[
            pl.BlockSpec((1, BS, D), lambda i, b: (b, i, 0)),
            pl.BlockSpec((BS, D), lambda i, b: (i, 0)),
        ],
        out_specs=pl.BlockSpec((1, BS, D), lambda i, b: (b, i, 0)),
        out_shape=jax.ShapeDtypeStruct(x.shape, x.dtype),
    )(x, pe)
